# baseline (speedup 1.0000x reference)
import functools

import jax
import jax.numpy as jnp
from jax import lax
from jax.experimental import pallas as pl
from jax.experimental.pallas import tpu as pltpu
from jax.experimental.pallas import tpu_sc as plsc

BATCH = 16384
EMBED_DIM = 32
TOTAL = 2 * BATCH

_info = plsc.get_sparse_core_info()
NC, NS, L = _info.num_cores, _info.num_subcores, _info.num_lanes
NW = NC * NS
B_PER_W = TOTAL // NW
CHUNK = 128
NCHUNK = B_PER_W // CHUNK
GROUPS = CHUNK // L

_mesh = plsc.VectorSubcoreMesh(core_axis_name="c", subcore_axis_name="s")


@functools.partial(
    pl.kernel,
    mesh=_mesh,
    out_type=jax.ShapeDtypeStruct((TOTAL,), jnp.float32),
    compiler_params=pltpu.CompilerParams(
        needs_layout_passes=False, use_tc_tiling_on_sc=False
    ),
    scratch_types=[
        pltpu.VMEM((NCHUNK, CHUNK), jnp.int32),
        pltpu.VMEM((NCHUNK, CHUNK), jnp.int32),
        pltpu.VMEM((NCHUNK, CHUNK), jnp.int32),
        pltpu.VMEM((CHUNK, EMBED_DIM), jnp.float32),
        pltpu.VMEM((CHUNK, EMBED_DIM), jnp.float32),
        pltpu.VMEM((CHUNK, EMBED_DIM), jnp.float32),
        pltpu.VMEM((CHUNK, EMBED_DIM), jnp.float32),
        pltpu.VMEM((CHUNK, EMBED_DIM), jnp.float32),
        pltpu.VMEM((CHUNK, EMBED_DIM), jnp.float32),
        pltpu.VMEM((B_PER_W,), jnp.float32),
        pltpu.VMEM((B_PER_W,), jnp.float32),
        pltpu.SemaphoreType.DMA,
    ],
)
def _complex_score_kernel(
    s_hbm, r_hbm, o_hbm,
    ent_real, ent_imag, rel_real, rel_imag,
    out_hbm,
    s_v, r_v, o_v,
    sr_v, si_v, or_v, oi_v, rr_v, ri_v,
    scores_v, acc_v, sem,
):
    wid = lax.axis_index("s") * NC + lax.axis_index("c")

    pltpu.sync_copy(s_hbm.at[wid], s_v)
    pltpu.sync_copy(r_hbm.at[wid], r_v)
    pltpu.sync_copy(o_hbm.at[wid], o_v)

    lane = lax.iota(jnp.int32, L)

    def chunk_body(g, carry):
        cps = [
            pltpu.async_copy(ent_real.at[s_v.at[g]], sr_v, sem),
            pltpu.async_copy(ent_imag.at[s_v.at[g]], si_v, sem),
            pltpu.async_copy(ent_real.at[o_v.at[g]], or_v, sem),
            pltpu.async_copy(ent_imag.at[o_v.at[g]], oi_v, sem),
            pltpu.async_copy(rel_real.at[r_v.at[g]], rr_v, sem),
            pltpu.async_copy(rel_imag.at[r_v.at[g]], ri_v, sem),
        ]
        for cp in cps:
            cp.wait()

        def zero_body(z, carry2):
            acc_v[pl.ds(g * CHUNK + z * L, L)] = jnp.zeros((L,), jnp.float32)
            return carry2

        lax.fori_loop(0, CHUNK // L, zero_body, 0)

        def trip_body(t, carry2):
            h = EMBED_DIM // 2
            sr0 = sr_v[t, pl.ds(0, h)]
            sr1 = sr_v[t, pl.ds(h, h)]
            si0 = si_v[t, pl.ds(0, h)]
            si1 = si_v[t, pl.ds(h, h)]
            or0 = or_v[t, pl.ds(0, h)]
            or1 = or_v[t, pl.ds(h, h)]
            oi0 = oi_v[t, pl.ds(0, h)]
            oi1 = oi_v[t, pl.ds(h, h)]
            rr0 = rr_v[t, pl.ds(0, h)]
            rr1 = rr_v[t, pl.ds(h, h)]
            ri0 = ri_v[t, pl.ds(0, h)]
            ri1 = ri_v[t, pl.ds(h, h)]
            v0 = sr0 * (or0 + oi0) + si0 * (oi0 - or0) + (rr0 + ri0)
            v1 = sr1 * (or1 + oi1) + si1 * (oi1 - or1) + (rr1 + ri1)
            v = v0 + v1
            slot = jnp.full((L,), g * CHUNK + t, jnp.int32)
            plsc.addupdate_scatter(acc_v, [slot], v)
            return carry2

        lax.fori_loop(0, CHUNK, trip_body, 0)
        return carry

    lax.fori_loop(0, NCHUNK, chunk_body, 0)

    def copy_body(z, carry):
        scores_v[pl.ds(z * L, L)] = acc_v[pl.ds(z * L, L)]
        return carry

    lax.fori_loop(0, B_PER_W // L, copy_body, 0)

    pltpu.sync_copy(scores_v, out_hbm.at[pl.ds(wid * B_PER_W, B_PER_W)])


def kernel(positive, negative, ent_real, ent_imag, rel_real, rel_imag):
    trip = jnp.concatenate([positive, negative], axis=0)
    s_idx = trip[:, 0].reshape(NW, NCHUNK, CHUNK)
    r_idx = trip[:, 1].reshape(NW, NCHUNK, CHUNK)
    o_idx = trip[:, 2].reshape(NW, NCHUNK, CHUNK)
    out = _complex_score_kernel(
        s_idx, r_idx, o_idx, ent_real, ent_imag, rel_real, rel_imag
    )
    return out[:BATCH], out[BATCH:]
